# baseline (speedup 1.0000x reference)
import jax
import jax.numpy as jnp
from jax import lax
from jax.experimental import pallas as pl
from jax.experimental.pallas import tpu as pltpu
from jax.experimental.pallas import tpu_sc as plsc

_EPS = 1e-09
_L_SEL, _L_ID, _L_RATE, _L_KL, _L_ENT, _L_SM = 1.0, 10.0, 5.0, 0.1, 0.05, 0.5

_T_BLK = 512


def _union_sc_body(candi_hbm, probs_hbm, first_hbm, pm_hbm,
                   cat_v, prob_v, frow_v, pmrow_v):
    b = lax.axis_index("s")
    core = lax.axis_index("c")
    f32 = jnp.float32

    @pl.when(core == 0)
    def _work():
        pltpu.sync_copy(candi_hbm.at[b], cat_v)
        pltpu.sync_copy(probs_hbm.at[b], prob_v)

        zero = jnp.zeros((16,), f32)
        ci = [cat_v[c // 4, pl.ds((c % 4) * 16, 16)] for c in range(8)]
        ivec = [lax.iota(jnp.int32, 16) + c * 16 for c in range(8)]
        init = (zero,) * 16

        def make_step(half):
            def step(jc, carry):
                pm_acc = list(carry[:8])
                dup_acc = list(carry[8:])
                off = jc * 16
                vb = cat_v[half, pl.ds(off, 16)]
                pb = prob_v[half, pl.ds(off, 16)]
                for k in range(16):
                    idxk = jnp.full((16,), k, jnp.int32)
                    catj = vb[idxk]
                    probj = pb[idxk]
                    jv = half * 64 + off + k
                    for c in range(8):
                        eqv = ci[c] == catj
                        pm_acc[c] = pm_acc[c] + jnp.where(eqv, probj, zero)
                        hit = jnp.where(eqv & (ivec[c] > jv), 1.0, 0.0)
                        dup_acc[c] = jnp.maximum(dup_acc[c], hit)
                return tuple(pm_acc) + tuple(dup_acc)
            return step

        res = lax.fori_loop(0, 4, make_step(0), init)
        res = lax.fori_loop(0, 4, make_step(1), res)

        for c in range(8):
            pmrow_v[pl.ds(c * 16, 16)] = res[c]
            frow_v[pl.ds(c * 16, 16)] = 1.0 - res[8 + c]

        pltpu.sync_copy(frow_v, first_hbm.at[b])
        pltpu.sync_copy(pmrow_v, pm_hbm.at[b])


def _union_sc(candi_ids, selector_probs):
    B, _, K = candi_ids.shape
    f32 = jnp.float32
    fn = pl.kernel(
        _union_sc_body,
        out_type=[jax.ShapeDtypeStruct((B, 2 * K), f32),
                  jax.ShapeDtypeStruct((B, 2 * K), f32)],
        mesh=plsc.VectorSubcoreMesh(core_axis_name="c", subcore_axis_name="s"),
        scratch_types=[
            pltpu.VMEM((2, K), jnp.int32),
            pltpu.VMEM((2, K), f32),
            pltpu.VMEM((2 * K,), f32),
            pltpu.VMEM((2 * K,), f32),
        ],
    )
    return fn(candi_ids, selector_probs)


def _stream_tc(out_ids_ref, lab_hbm, rates_hbm, trg_rates_hbm,
               sel_probs_ref, sel_onehot_ref, lens_ref,
               nll_ref, mask_ref, misc_ref, dist_ref,
               lab_buf, lab_sem, rates_buf, tr_buf, small_sem):
    i = pl.program_id(0)
    nt = pl.num_programs(0)
    slot = jax.lax.rem(i, 2)
    T = rates_buf.shape[0]

    @pl.when(i == 0)
    def _init():
        nll_ref[0, 0] = 0.0
        mask_ref[0, 0] = 0.0
        dist_ref[...] = jnp.zeros_like(dist_ref)
        pltpu.make_async_copy(lab_hbm.at[pl.ds(1, _T_BLK)],
                              lab_buf.at[0], lab_sem.at[0]).start()
        pltpu.make_async_copy(rates_hbm.at[:, 0, :],
                              rates_buf, small_sem.at[0]).start()
        pltpu.make_async_copy(trg_rates_hbm.at[pl.ds(1, T), 0, :],
                              tr_buf, small_sem.at[1]).start()

    @pl.when(i + 1 < nt)
    def _prefetch():
        nxt = jax.lax.rem(i + 1, 2)
        pltpu.make_async_copy(lab_hbm.at[pl.ds((i + 1) * _T_BLK + 1, _T_BLK)],
                              lab_buf.at[nxt], lab_sem.at[nxt]).start()

    pltpu.make_async_copy(lab_hbm.at[pl.ds(i * _T_BLK + 1, _T_BLK)],
                          lab_buf.at[slot], lab_sem.at[slot]).wait()

    out = out_ids_ref[...]
    lab = lab_buf[slot]
    p_true = jnp.clip((out * lab).sum(axis=-1), _EPS)
    step_mask = (lab.sum(axis=-1) > 0.5).astype(jnp.float32)
    nll_ref[0, 0] += (-jnp.log(p_true) * step_mask).sum()
    mask_ref[0, 0] += step_mask.sum()
    dist_ref[...] += out.sum(axis=0)

    @pl.when(i == 1)
    def _small_terms():
        lens = lens_ref[...]
        f32 = jnp.float32

        probs = sel_probs_ref[...]
        onehot = sel_onehot_ref[...]
        sel_mask = (onehot.sum(axis=-1) > 0.5).astype(f32)
        sel_p_true = jnp.clip((probs * onehot).sum(axis=-1), _EPS)
        nll_sel = -jnp.log(sel_p_true) * sel_mask
        loss_sel = nll_sel.sum() / jnp.clip(sel_mask.sum(), 1.0)

        pc = jnp.clip(probs, _EPS)
        ent_sum = 0.5 * (-(pc * jnp.log(pc)).sum())
        bs = f32(probs.shape[0])
        loss_ent = _L_ENT * ent_sum / bs

        pltpu.make_async_copy(rates_hbm.at[:, 0, :],
                              rates_buf, small_sem.at[0]).wait()
        pltpu.make_async_copy(trg_rates_hbm.at[pl.ds(1, T), 0, :],
                              tr_buf, small_sem.at[1]).wait()
        r = rates_buf[...]
        tr = tr_buf[...]
        denom_rate = jnp.maximum(1, (lens - 2).sum()).astype(f32)
        loss_rate = jnp.abs(r - tr).sum() * _L_RATE / denom_rate

        dr = jnp.abs(r[1:] - r[:-1])
        eff = jnp.maximum(lens - 3, 0)
        t_iota = jax.lax.broadcasted_iota(jnp.int32, dr.shape, 0)
        sm_mask = (t_iota < eff).astype(f32)
        loss_smooth = _L_SM * (dr * sm_mask).sum()

        misc_ref[0, 0] = (_L_SEL * loss_sel + loss_rate
                          + loss_ent + loss_smooth)


def _fin_tc(nll_ref, mask_ref, misc_ref, dist_ref, first_ref, pm_ref,
            candi_ref, routes_ref, sel_probs_ref, total_ref):
    f32 = jnp.float32
    candi = candi_ref[...]
    cat = jnp.concatenate([candi[:, 0, :], candi[:, 1, :]], axis=-1)
    routes = routes_ref[...]
    dist = dist_ref[...]
    first = first_ref[...]
    bs = f32(cat.shape[0])

    matches = (routes[:, :, None] == cat[:, None, :]).astype(f32)
    agg = (matches * dist[:, :, None]).sum(axis=1)
    s = jnp.clip((agg * first).sum(axis=-1), _EPS)
    pb = agg / s[:, None]
    pb_c = jnp.clip(pb, _EPS)
    psum = jnp.clip(sel_probs_ref[...].sum(axis=(1, 2)), _EPS)
    lpm = jnp.log(jnp.clip(pm_ref[...] / psum[:, None], _EPS))
    kl = (first * pb_c * (jnp.log(pb_c) - lpm)).sum()
    loss_kl = _L_KL * kl / bs

    loss_id = nll_ref[0, 0] * _L_ID / jnp.clip(mask_ref[0, 0], 1.0)
    total_ref[0, 0] = misc_ref[0, 0] + loss_id + loss_kl


def kernel(selector_logits, selector_probs, out_ids, out_rates, selector_onehot,
           trg_labels, trg_rates, candi_ids, routes, trg_lengths):
    T, B, R = out_ids.shape
    K = candi_ids.shape[2]
    f32 = jnp.float32
    lens2 = trg_lengths.reshape(1, B)
    rates3 = out_rates.reshape(T, 1, B)
    tr3 = trg_rates.reshape(trg_rates.shape[0], 1, B)

    first, pm = _union_sc(candi_ids, selector_probs)

    nt = T // _T_BLK
    big = pl.BlockSpec((_T_BLK, B, R), lambda i: (i, 0, 0))
    full = lambda shp: pl.BlockSpec(shp, lambda i: (0,) * len(shp))
    smem_out = pl.BlockSpec(memory_space=pltpu.SMEM)

    nll, mask, misc, dist = pl.pallas_call(
        _stream_tc,
        grid=(nt,),
        in_specs=[
            big,
            pl.BlockSpec(memory_space=pl.ANY),
            pl.BlockSpec(memory_space=pl.ANY),
            pl.BlockSpec(memory_space=pl.ANY),
            full(selector_probs.shape), full(selector_onehot.shape),
            full((1, B)),
        ],
        out_specs=[smem_out, smem_out, smem_out,
                   pl.BlockSpec((B, R), lambda i: (0, 0))],
        out_shape=[jax.ShapeDtypeStruct((1, 1), f32),
                   jax.ShapeDtypeStruct((1, 1), f32),
                   jax.ShapeDtypeStruct((1, 1), f32),
                   jax.ShapeDtypeStruct((B, R), f32)],
        scratch_shapes=[
            pltpu.VMEM((2, _T_BLK, B, R), f32),
            pltpu.SemaphoreType.DMA((2,)),
            pltpu.VMEM((T, B), f32),
            pltpu.VMEM((T, B), f32),
            pltpu.SemaphoreType.DMA((2,)),
        ],
        compiler_params=pltpu.CompilerParams(
            dimension_semantics=("arbitrary",)),
    )(out_ids, trg_labels, rates3, tr3, selector_probs,
      selector_onehot, lens2)

    smem_in = pl.BlockSpec(memory_space=pltpu.SMEM)
    full0 = lambda shp: pl.BlockSpec(shp, lambda: (0,) * len(shp))
    total = pl.pallas_call(
        _fin_tc,
        in_specs=[smem_in, smem_in, smem_in,
                  full0((B, R)), full0((B, 2 * K)), full0((B, 2 * K)),
                  full0((B, 2, K)), full0((B, R)), full0((B, 2, K))],
        out_specs=smem_out,
        out_shape=jax.ShapeDtypeStruct((1, 1), f32),
    )(nll, mask, misc, dist, first, pm, candi_ids, routes, selector_probs)
    return total[0, 0]


# speedup vs baseline: 1.0161x; 1.0161x over previous
import jax
import jax.numpy as jnp
from jax import lax
from jax.experimental import pallas as pl
from jax.experimental.pallas import tpu as pltpu
from jax.experimental.pallas import tpu_sc as plsc

_EPS = 1e-09
_L_SEL, _L_ID, _L_RATE, _L_KL, _L_ENT, _L_SM = 1.0, 10.0, 5.0, 0.1, 0.05, 0.5

_T_BLK = 512


def _union_sc_body(candi_hbm, probs_hbm, first_hbm, pm_hbm,
                   cat_v, prob_v, frow_v, pmrow_v):
    b = lax.axis_index("s")
    core = lax.axis_index("c")
    f32 = jnp.float32

    pltpu.sync_copy(candi_hbm.at[b], cat_v)
    zero = jnp.zeros((16,), f32)
    ci = [cat_v[c // 4, pl.ds((c % 4) * 16, 16)] for c in range(8)]
    init = (zero,) * 8

    @pl.when(core == 0)
    def _pm_work():
        pltpu.sync_copy(probs_hbm.at[b], prob_v)

        def make_step(half):
            def step(jc, carry):
                pm_acc = list(carry)
                off = jc * 16
                vb = cat_v[half, pl.ds(off, 16)]
                pb = prob_v[half, pl.ds(off, 16)]
                for k in range(16):
                    idxk = jnp.full((16,), k, jnp.int32)
                    catj = vb[idxk]
                    probj = pb[idxk]
                    for c in range(8):
                        pm_acc[c] = pm_acc[c] + jnp.where(
                            ci[c] == catj, probj, zero)
                return tuple(pm_acc)
            return step

        res = lax.fori_loop(0, 4, make_step(0), init)
        res = lax.fori_loop(0, 4, make_step(1), res)
        for c in range(8):
            pmrow_v[pl.ds(c * 16, 16)] = res[c]
        pltpu.sync_copy(pmrow_v, pm_hbm.at[b])

    @pl.when(core == 1)
    def _dup_work():
        ivec = [lax.iota(jnp.int32, 16) + c * 16 for c in range(8)]

        def make_step(half):
            def step(jc, carry):
                dup_acc = list(carry)
                off = jc * 16
                vb = cat_v[half, pl.ds(off, 16)]
                for k in range(16):
                    idxk = jnp.full((16,), k, jnp.int32)
                    catj = vb[idxk]
                    jv = half * 64 + off + k
                    for c in range(4 * half, 8):
                        hit = jnp.where(
                            (ci[c] == catj) & (ivec[c] > jv), 1.0, 0.0)
                        dup_acc[c] = jnp.maximum(dup_acc[c], hit)
                return tuple(dup_acc)
            return step

        res = lax.fori_loop(0, 4, make_step(0), init)
        res = lax.fori_loop(0, 4, make_step(1), res)
        for c in range(8):
            frow_v[pl.ds(c * 16, 16)] = 1.0 - res[c]
        pltpu.sync_copy(frow_v, first_hbm.at[b])


def _union_sc(candi_ids, selector_probs):
    B, _, K = candi_ids.shape
    f32 = jnp.float32
    fn = pl.kernel(
        _union_sc_body,
        out_type=[jax.ShapeDtypeStruct((B, 2 * K), f32),
                  jax.ShapeDtypeStruct((B, 2 * K), f32)],
        mesh=plsc.VectorSubcoreMesh(core_axis_name="c", subcore_axis_name="s"),
        scratch_types=[
            pltpu.VMEM((2, K), jnp.int32),
            pltpu.VMEM((2, K), f32),
            pltpu.VMEM((2 * K,), f32),
            pltpu.VMEM((2 * K,), f32),
        ],
    )
    return fn(candi_ids, selector_probs)


def _stream_tc(out_ids_ref, lab_hbm, rates_hbm, trg_rates_hbm,
               sel_probs_ref, sel_onehot_ref, lens_ref,
               nll_ref, mask_ref, misc_ref, dist_ref,
               lab_buf, lab_sem, rates_buf, tr_buf, small_sem):
    i = pl.program_id(0)
    nt = pl.num_programs(0)
    slot = jax.lax.rem(i, 2)
    T = rates_buf.shape[0]

    @pl.when(i == 0)
    def _init():
        nll_ref[0, 0] = 0.0
        mask_ref[0, 0] = 0.0
        dist_ref[...] = jnp.zeros_like(dist_ref)
        pltpu.make_async_copy(lab_hbm.at[pl.ds(1, _T_BLK)],
                              lab_buf.at[0], lab_sem.at[0]).start()
        pltpu.make_async_copy(rates_hbm.at[:, 0, :],
                              rates_buf, small_sem.at[0]).start()
        pltpu.make_async_copy(trg_rates_hbm.at[pl.ds(1, T), 0, :],
                              tr_buf, small_sem.at[1]).start()

    @pl.when(i + 1 < nt)
    def _prefetch():
        nxt = jax.lax.rem(i + 1, 2)
        pltpu.make_async_copy(lab_hbm.at[pl.ds((i + 1) * _T_BLK + 1, _T_BLK)],
                              lab_buf.at[nxt], lab_sem.at[nxt]).start()

    pltpu.make_async_copy(lab_hbm.at[pl.ds(i * _T_BLK + 1, _T_BLK)],
                          lab_buf.at[slot], lab_sem.at[slot]).wait()

    out = out_ids_ref[...]
    lab = lab_buf[slot]
    p_true = jnp.clip((out * lab).sum(axis=-1), _EPS)
    step_mask = (lab.sum(axis=-1) > 0.5).astype(jnp.float32)
    nll_ref[0, 0] += (-jnp.log(p_true) * step_mask).sum()
    mask_ref[0, 0] += step_mask.sum()
    dist_ref[...] += out.sum(axis=0)

    @pl.when(i == 1)
    def _small_terms():
        lens = lens_ref[...]
        f32 = jnp.float32

        probs = sel_probs_ref[...]
        onehot = sel_onehot_ref[...]
        sel_mask = (onehot.sum(axis=-1) > 0.5).astype(f32)
        sel_p_true = jnp.clip((probs * onehot).sum(axis=-1), _EPS)
        nll_sel = -jnp.log(sel_p_true) * sel_mask
        loss_sel = nll_sel.sum() / jnp.clip(sel_mask.sum(), 1.0)

        pc = jnp.clip(probs, _EPS)
        ent_sum = 0.5 * (-(pc * jnp.log(pc)).sum())
        bs = f32(probs.shape[0])
        loss_ent = _L_ENT * ent_sum / bs

        pltpu.make_async_copy(rates_hbm.at[:, 0, :],
                              rates_buf, small_sem.at[0]).wait()
        pltpu.make_async_copy(trg_rates_hbm.at[pl.ds(1, T), 0, :],
                              tr_buf, small_sem.at[1]).wait()
        r = rates_buf[...]
        tr = tr_buf[...]
        denom_rate = jnp.maximum(1, (lens - 2).sum()).astype(f32)
        loss_rate = jnp.abs(r - tr).sum() * _L_RATE / denom_rate

        dr = jnp.abs(r[1:] - r[:-1])
        eff = jnp.maximum(lens - 3, 0)
        t_iota = jax.lax.broadcasted_iota(jnp.int32, dr.shape, 0)
        sm_mask = (t_iota < eff).astype(f32)
        loss_smooth = _L_SM * (dr * sm_mask).sum()

        misc_ref[0, 0] = (_L_SEL * loss_sel + loss_rate
                          + loss_ent + loss_smooth)


def _fin_tc(nll_ref, mask_ref, misc_ref, dist_ref, first_ref, pm_ref,
            candi_ref, routes_ref, sel_probs_ref, total_ref):
    f32 = jnp.float32
    candi = candi_ref[...]
    cat = jnp.concatenate([candi[:, 0, :], candi[:, 1, :]], axis=-1)
    routes = routes_ref[...]
    dist = dist_ref[...]
    first = first_ref[...]
    bs = f32(cat.shape[0])

    matches = (routes[:, :, None] == cat[:, None, :]).astype(f32)
    agg = (matches * dist[:, :, None]).sum(axis=1)
    s = jnp.clip((agg * first).sum(axis=-1), _EPS)
    pb = agg / s[:, None]
    pb_c = jnp.clip(pb, _EPS)
    psum = jnp.clip(sel_probs_ref[...].sum(axis=(1, 2)), _EPS)
    lpm = jnp.log(jnp.clip(pm_ref[...] / psum[:, None], _EPS))
    kl = (first * pb_c * (jnp.log(pb_c) - lpm)).sum()
    loss_kl = _L_KL * kl / bs

    loss_id = nll_ref[0, 0] * _L_ID / jnp.clip(mask_ref[0, 0], 1.0)
    total_ref[0, 0] = misc_ref[0, 0] + loss_id + loss_kl


def kernel(selector_logits, selector_probs, out_ids, out_rates, selector_onehot,
           trg_labels, trg_rates, candi_ids, routes, trg_lengths):
    T, B, R = out_ids.shape
    K = candi_ids.shape[2]
    f32 = jnp.float32
    lens2 = trg_lengths.reshape(1, B)
    rates3 = out_rates.reshape(T, 1, B)
    tr3 = trg_rates.reshape(trg_rates.shape[0], 1, B)

    first, pm = _union_sc(candi_ids, selector_probs)

    nt = T // _T_BLK
    big = pl.BlockSpec((_T_BLK, B, R), lambda i: (i, 0, 0))
    full = lambda shp: pl.BlockSpec(shp, lambda i: (0,) * len(shp))
    smem_out = pl.BlockSpec(memory_space=pltpu.SMEM)

    nll, mask, misc, dist = pl.pallas_call(
        _stream_tc,
        grid=(nt,),
        in_specs=[
            big,
            pl.BlockSpec(memory_space=pl.ANY),
            pl.BlockSpec(memory_space=pl.ANY),
            pl.BlockSpec(memory_space=pl.ANY),
            full(selector_probs.shape), full(selector_onehot.shape),
            full((1, B)),
        ],
        out_specs=[smem_out, smem_out, smem_out,
                   pl.BlockSpec((B, R), lambda i: (0, 0))],
        out_shape=[jax.ShapeDtypeStruct((1, 1), f32),
                   jax.ShapeDtypeStruct((1, 1), f32),
                   jax.ShapeDtypeStruct((1, 1), f32),
                   jax.ShapeDtypeStruct((B, R), f32)],
        scratch_shapes=[
            pltpu.VMEM((2, _T_BLK, B, R), f32),
            pltpu.SemaphoreType.DMA((2,)),
            pltpu.VMEM((T, B), f32),
            pltpu.VMEM((T, B), f32),
            pltpu.SemaphoreType.DMA((2,)),
        ],
        compiler_params=pltpu.CompilerParams(
            dimension_semantics=("arbitrary",)),
    )(out_ids, trg_labels, rates3, tr3, selector_probs,
      selector_onehot, lens2)

    smem_in = pl.BlockSpec(memory_space=pltpu.SMEM)
    full0 = lambda shp: pl.BlockSpec(shp, lambda: (0,) * len(shp))
    total = pl.pallas_call(
        _fin_tc,
        in_specs=[smem_in, smem_in, smem_in,
                  full0((B, R)), full0((B, 2 * K)), full0((B, 2 * K)),
                  full0((B, 2, K)), full0((B, R)), full0((B, 2, K))],
        out_specs=smem_out,
        out_shape=jax.ShapeDtypeStruct((1, 1), f32),
    )(nll, mask, misc, dist, first, pm, candi_ids, routes, selector_probs)
    return total[0, 0]
